# baseline (speedup 1.0000x reference)
import functools

import jax
import jax.numpy as jnp
from jax import lax
from jax.experimental import pallas as pl
from jax.experimental.pallas import tpu as pltpu
from jax.experimental.pallas import tpu_sc as plsc

_B, _S = 16384, 200
_VOCAB = 1_000_000

_NC, _NS = 2, 16
_NW = _NC * _NS
_ROWS = _B // _NW
_CROWS = 32
_NCHUNK = _ROWS // _CROWS

_COLS = tuple(range(0, _S - 16, 16)) + (_S - 16,)
_RSTRIDE = 16 * len(_COLS)
_CH = _CROWS * _RSTRIDE

_TBL_CH = 62_400
_TBL_P = _TBL_CH // 10
_TBL_TAIL = _VOCAB - _NS * _TBL_CH

_mesh = plsc.VectorSubcoreMesh(core_axis_name="c", subcore_axis_name="s")


@functools.partial(
    pl.kernel,
    mesh=_mesh,
    out_type=jax.ShapeDtypeStruct((_B, _S), jnp.int32),
    compiler_params=pltpu.CompilerParams(needs_layout_passes=False),
    scratch_types=[
        pltpu.VMEM_SHARED((_VOCAB,), jnp.int32),
        pltpu.VMEM((_CROWS, _S), jnp.int32),
        pltpu.VMEM((_CROWS, _S), jnp.int32),
        pltpu.VMEM((_CROWS, _S), jnp.int32),
        pltpu.VMEM((_CROWS, _S), jnp.int32),
        pltpu.VMEM((_CH,), jnp.int32),
        pltpu.VMEM((_CH,), jnp.int32),
        pltpu.VMEM((_CH,), jnp.int32),
        pltpu.VMEM((_CH,), jnp.int32),
        pltpu.VMEM((_CH,), jnp.int32),
        pltpu.SemaphoreType.DMA,
        pltpu.SemaphoreType.DMA,
        pltpu.SemaphoreType.DMA,
        pltpu.SemaphoreType.DMA,
        pltpu.SemaphoreType.DMA,
        pltpu.SemaphoreType.DMA,
        pltpu.SemaphoreType.DMA,
    ],
)
def _lookup(ids_hbm, map_hbm, out_hbm, tbl_sh,
            idx2d0, idx2d1, val2d0, val2d1, ilin0, ilin1,
            vlin0, vlin1, vlin2,
            si0, si1, sg0, sg1, sg2, so0, so1):
    cid = lax.axis_index("c")
    sid = lax.axis_index("s")
    wid = sid * _NC + cid
    r0 = wid * _ROWS

    idx2d = (idx2d0, idx2d1)
    val2d = (val2d0, val2d1)
    ilin = (ilin0, ilin1)
    vlin = (vlin0, vlin1, vlin2)
    sem_i = (si0, si1)
    sem_g = (sg0, sg1, sg2)
    sem_o = (so0, so1)

    cp_in = [None] * _NCHUNK
    for k in range(2):
        cp_in[k] = pltpu.async_copy(
            ids_hbm.at[pl.ds(r0 + k * _CROWS, _CROWS)], idx2d[k], sem_i[k])

    for p in range(10):
        toff = sid * _TBL_CH + p * _TBL_P
        pltpu.sync_copy(map_hbm.at[pl.ds(toff, _TBL_P)],
                        vlin0.at[pl.ds(0, _TBL_P)])
        pltpu.sync_copy(vlin0.at[pl.ds(0, _TBL_P)],
                        tbl_sh.at[pl.ds(toff, _TBL_P)])

    @pl.when(sid == _NS - 1)
    def _copy_tail():
        pltpu.sync_copy(
            map_hbm.at[pl.ds(_NS * _TBL_CH, _TBL_TAIL)],
            vlin0.at[pl.ds(0, _TBL_TAIL)],
        )
        pltpu.sync_copy(
            vlin0.at[pl.ds(0, _TBL_TAIL)],
            tbl_sh.at[pl.ds(_NS * _TBL_CH, _TBL_TAIL)],
        )

    plsc.subcore_barrier()

    def _make_fused(src2d, dst_lin, exp_lin, exp2d):
        def body(i, carry):
            base = i * _RSTRIDE
            for j, c in enumerate(_COLS):
                dst_lin[pl.ds(base + j * 16, 16)] = src2d[i, pl.ds(c, 16)]
                if exp_lin is not None:
                    exp2d[i, pl.ds(c, 16)] = exp_lin[pl.ds(base + j * 16, 16)]
            return carry
        return body

    g_cp = {}
    out_cp = {}
    for k in range(_NCHUNK):
        b = k % 2
        cp_in[k].wait()
        if k >= 2:
            g_cp[k - 2].wait()
            if k >= 4:
                out_cp[k - 4].wait()
            body = _make_fused(idx2d[b], ilin[b],
                               vlin[(k - 2) % 3], val2d[(k - 2) % 2])
        else:
            body = _make_fused(idx2d[b], ilin[b], None, None)
        lax.fori_loop(0, _CROWS, body, 0)
        g_cp[k] = pltpu.async_copy(tbl_sh.at[ilin[b]], vlin[k % 3],
                                   sem_g[k % 3])
        if k + 2 < _NCHUNK:
            cp_in[k + 2] = pltpu.async_copy(
                ids_hbm.at[pl.ds(r0 + (k + 2) * _CROWS, _CROWS)],
                idx2d[b], sem_i[b])
        if k >= 2:
            out_cp[k - 2] = pltpu.async_copy(
                val2d[(k - 2) % 2],
                out_hbm.at[pl.ds(r0 + (k - 2) * _CROWS, _CROWS)],
                sem_o[(k - 2) % 2])

    def _make_expand(exp_lin, exp2d):
        def body(i, carry):
            base = i * _RSTRIDE
            for j, c in enumerate(_COLS):
                exp2d[i, pl.ds(c, 16)] = exp_lin[pl.ds(base + j * 16, 16)]
            return carry
        return body

    for t in (_NCHUNK - 2, _NCHUNK - 1):
        g_cp[t].wait()
        out_cp[t - 2].wait()
        lax.fori_loop(0, _CROWS, _make_expand(vlin[t % 3], val2d[t % 2]), 0)
        out_cp[t] = pltpu.async_copy(
            val2d[t % 2],
            out_hbm.at[pl.ds(r0 + t * _CROWS, _CROWS)], sem_o[t % 2])
    out_cp[_NCHUNK - 2].wait()
    out_cp[_NCHUNK - 1].wait()


def kernel(token_ids, mapping):
    return _lookup(token_ids, mapping)


# speedup vs baseline: 1.1209x; 1.1209x over previous
import functools

import jax
import jax.numpy as jnp
from jax import lax
from jax.experimental import pallas as pl
from jax.experimental.pallas import tpu as pltpu
from jax.experimental.pallas import tpu_sc as plsc

_B, _S = 16384, 200
_VOCAB = 1_000_000

_NC, _NS = 2, 16
_NW = _NC * _NS
_COLS = _B // _NW
_CR = 8
_NCHUNK = _S // _CR
_CH = _CR * _COLS
_NG = _COLS // 16

_TBL_CH = 62_464
_TBL_P = _TBL_CH // 16
_TBL_TAIL = _VOCAB - _NS * _TBL_CH

_mesh = plsc.VectorSubcoreMesh(core_axis_name="c", subcore_axis_name="s")


@functools.partial(
    pl.kernel,
    mesh=_mesh,
    out_type=jax.ShapeDtypeStruct((_S, _B), jnp.int32),
    compiler_params=pltpu.CompilerParams(needs_layout_passes=False),
    scratch_types=[
        pltpu.VMEM_SHARED((_VOCAB,), jnp.int32),
        pltpu.VMEM((_CR, _COLS), jnp.int32),
        pltpu.VMEM((_CR, _COLS), jnp.int32),
        pltpu.VMEM((_CR, _COLS), jnp.int32),
        pltpu.VMEM((_CR, _COLS), jnp.int32),
        pltpu.VMEM((_CH,), jnp.int32),
        pltpu.VMEM((_CH,), jnp.int32),
        pltpu.VMEM((_CH,), jnp.int32),
        pltpu.VMEM((_CH,), jnp.int32),
        pltpu.VMEM((_CH,), jnp.int32),
        pltpu.SemaphoreType.DMA,
        pltpu.SemaphoreType.DMA,
        pltpu.SemaphoreType.DMA,
        pltpu.SemaphoreType.DMA,
        pltpu.SemaphoreType.DMA,
        pltpu.SemaphoreType.DMA,
        pltpu.SemaphoreType.DMA,
    ],
)
def _lookup(ids_hbm, map_hbm, out_hbm, tbl_sh,
            idx2d0, idx2d1, val2d0, val2d1, ilin0, ilin1,
            vlin0, vlin1, vlin2,
            si0, si1, sg0, sg1, sg2, so0, so1):
    cid = lax.axis_index("c")
    sid = lax.axis_index("s")
    wid = sid * _NC + cid
    c0 = wid * _COLS

    idx2d = (idx2d0, idx2d1)
    val2d = (val2d0, val2d1)
    ilin = (ilin0, ilin1)
    vlin = (vlin0, vlin1, vlin2)
    sem_i = (si0, si1)
    sem_g = (sg0, sg1, sg2)
    sem_o = (so0, so1)

    cp_in = [None] * _NCHUNK
    for k in range(2):
        cp_in[k] = pltpu.async_copy(
            ids_hbm.at[pl.ds(k * _CR, _CR), pl.ds(c0, _COLS)],
            idx2d[k], sem_i[k])

    for p in range(16):
        toff = sid * _TBL_CH + p * _TBL_P
        pltpu.sync_copy(map_hbm.at[pl.ds(toff, _TBL_P)],
                        vlin0.at[pl.ds(0, _TBL_P)])
        pltpu.sync_copy(vlin0.at[pl.ds(0, _TBL_P)],
                        tbl_sh.at[pl.ds(toff, _TBL_P)])

    @pl.when(sid == _NS - 1)
    def _copy_tail():
        pltpu.sync_copy(
            map_hbm.at[pl.ds(_NS * _TBL_CH, _TBL_TAIL)],
            vlin0.at[pl.ds(0, _TBL_TAIL)],
        )
        pltpu.sync_copy(
            vlin0.at[pl.ds(0, _TBL_TAIL)],
            tbl_sh.at[pl.ds(_NS * _TBL_CH, _TBL_TAIL)],
        )

    plsc.subcore_barrier()

    def _make_fused(src2d, dst_lin, exp_lin, exp2d):
        def body(q, carry):
            for u in range(4):
                gg = q * 4 + u
                row = gg >> 5
                c = (gg & 31) * 16
                off = gg * 16
                dst_lin[pl.ds(off, 16)] = src2d[row, pl.ds(c, 16)]
                if exp_lin is not None:
                    exp2d[row, pl.ds(c, 16)] = exp_lin[pl.ds(off, 16)]
            return carry
        return body

    g_cp = {}
    out_cp = {}
    for k in range(_NCHUNK):
        b = k % 2
        cp_in[k].wait()
        if k >= 2:
            g_cp[k - 2].wait()
            if k >= 4:
                out_cp[k - 4].wait()
            body = _make_fused(idx2d[b], ilin[b],
                               vlin[(k - 2) % 3], val2d[(k - 2) % 2])
        else:
            body = _make_fused(idx2d[b], ilin[b], None, None)
        lax.fori_loop(0, _CR * _NG // 4, body, 0)
        g_cp[k] = pltpu.async_copy(tbl_sh.at[ilin[b]], vlin[k % 3],
                                   sem_g[k % 3])
        if k + 2 < _NCHUNK:
            cp_in[k + 2] = pltpu.async_copy(
                ids_hbm.at[pl.ds((k + 2) * _CR, _CR), pl.ds(c0, _COLS)],
                idx2d[b], sem_i[b])
        if k >= 2:
            out_cp[k - 2] = pltpu.async_copy(
                val2d[(k - 2) % 2],
                out_hbm.at[pl.ds((k - 2) * _CR, _CR), pl.ds(c0, _COLS)],
                sem_o[(k - 2) % 2])

    def _make_expand(exp_lin, exp2d):
        def body(q, carry):
            for u in range(4):
                gg = q * 4 + u
                row = gg >> 5
                c = (gg & 31) * 16
                off = gg * 16
                exp2d[row, pl.ds(c, 16)] = exp_lin[pl.ds(off, 16)]
            return carry
        return body

    for t in (_NCHUNK - 2, _NCHUNK - 1):
        g_cp[t].wait()
        out_cp[t - 2].wait()
        lax.fori_loop(0, _CR * _NG // 4,
                      _make_expand(vlin[t % 3], val2d[t % 2]), 0)
        out_cp[t] = pltpu.async_copy(
            val2d[t % 2],
            out_hbm.at[pl.ds(t * _CR, _CR), pl.ds(c0, _COLS)],
            sem_o[t % 2])
    out_cp[_NCHUNK - 2].wait()
    out_cp[_NCHUNK - 1].wait()


def kernel(token_ids, mapping):
    return _lookup(token_ids.T, mapping).T


# speedup vs baseline: 1.2394x; 1.1057x over previous
import functools

import jax
import jax.numpy as jnp
from jax import lax
from jax.experimental import pallas as pl
from jax.experimental.pallas import tpu as pltpu
from jax.experimental.pallas import tpu_sc as plsc

_B, _S = 16384, 200
_VOCAB = 1_000_000

_NC, _NS = 2, 16
_NW = _NC * _NS
_COLS = _B // _NW
_CR = 8
_NCHUNK = _S // _CR
_CH = _CR * _COLS
_NG = _COLS // 16

_TBL_CH = 62_464
_TBL_P = _TBL_CH // 16
_TBL_TAIL = _VOCAB - _NS * _TBL_CH

_mesh = plsc.VectorSubcoreMesh(core_axis_name="c", subcore_axis_name="s")


@functools.partial(
    pl.kernel,
    mesh=_mesh,
    out_type=jax.ShapeDtypeStruct((_S, _B), jnp.int32),
    compiler_params=pltpu.CompilerParams(needs_layout_passes=False),
    scratch_types=[
        pltpu.VMEM_SHARED((_VOCAB,), jnp.int32),
        pltpu.VMEM((_CR, _COLS), jnp.int32),
        pltpu.VMEM((_CR, _COLS), jnp.int32),
        pltpu.VMEM((_CR, _COLS), jnp.int32),
        pltpu.VMEM((_CR, _COLS), jnp.int32),
        pltpu.VMEM((_CH,), jnp.int32),
        pltpu.VMEM((_CH,), jnp.int32),
        pltpu.VMEM((_CH,), jnp.int32),
        pltpu.VMEM((_CH,), jnp.int32),
        pltpu.VMEM((_CH,), jnp.int32),
        pltpu.SemaphoreType.DMA,
        pltpu.SemaphoreType.DMA,
        pltpu.SemaphoreType.DMA,
        pltpu.SemaphoreType.DMA,
        pltpu.SemaphoreType.DMA,
        pltpu.SemaphoreType.DMA,
        pltpu.SemaphoreType.DMA,
    ],
)
def _lookup(ids_hbm, map_hbm, out_hbm, tbl_sh,
            idx2d0, idx2d1, val2d0, val2d1, ilin0, ilin1,
            vlin0, vlin1, vlin2,
            si0, si1, sg0, sg1, sg2, so0, so1):
    cid = lax.axis_index("c")
    sid = lax.axis_index("s")
    wid = sid * _NC + cid
    c0 = wid * _COLS

    idx2d = (idx2d0, idx2d1)
    val2d = (val2d0, val2d1)
    ilin = (ilin0, ilin1)
    vlin = (vlin0, vlin1, vlin2)
    sem_i = (si0, si1)
    sem_g = (sg0, sg1, sg2)
    sem_o = (so0, so1)

    cp_in = [None] * _NCHUNK
    for k in range(2):
        cp_in[k] = pltpu.async_copy(
            ids_hbm.at[pl.ds(k * _CR, _CR), pl.ds(c0, _COLS)],
            idx2d[k], sem_i[k])

    for p in range(16):
        toff = sid * _TBL_CH + p * _TBL_P
        pltpu.sync_copy(map_hbm.at[pl.ds(toff, _TBL_P)],
                        vlin0.at[pl.ds(0, _TBL_P)])
        pltpu.sync_copy(vlin0.at[pl.ds(0, _TBL_P)],
                        tbl_sh.at[pl.ds(toff, _TBL_P)])

    @pl.when(sid == _NS - 1)
    def _copy_tail():
        pltpu.sync_copy(
            map_hbm.at[pl.ds(_NS * _TBL_CH, _TBL_TAIL)],
            vlin0.at[pl.ds(0, _TBL_TAIL)],
        )
        pltpu.sync_copy(
            vlin0.at[pl.ds(0, _TBL_TAIL)],
            tbl_sh.at[pl.ds(_NS * _TBL_CH, _TBL_TAIL)],
        )

    plsc.subcore_barrier()

    def _make_fused(src2d, dst_lin, exp_lin, exp2d):
        def body(q, carry):
            row = q >> 2
            cb = (q & 3) * 128
            ob = q * 128
            for u in range(8):
                c = cb + u * 16
                off = ob + u * 16
                dst_lin[pl.ds(off, 16)] = src2d[row, pl.ds(c, 16)]
                if exp_lin is not None:
                    exp2d[row, pl.ds(c, 16)] = exp_lin[pl.ds(off, 16)]
            return carry
        return body

    g_cp = {}
    out_cp = {}
    for k in range(_NCHUNK):
        b = k % 2
        cp_in[k].wait()
        if k >= 2:
            g_cp[k - 2].wait()
            if k >= 4:
                out_cp[k - 4].wait()
            body = _make_fused(idx2d[b], ilin[b],
                               vlin[(k - 2) % 3], val2d[(k - 2) % 2])
        else:
            body = _make_fused(idx2d[b], ilin[b], None, None)
        lax.fori_loop(0, _CR * _NG // 8, body, 0)
        g_cp[k] = pltpu.async_copy(tbl_sh.at[ilin[b]], vlin[k % 3],
                                   sem_g[k % 3])
        if k + 2 < _NCHUNK:
            cp_in[k + 2] = pltpu.async_copy(
                ids_hbm.at[pl.ds((k + 2) * _CR, _CR), pl.ds(c0, _COLS)],
                idx2d[b], sem_i[b])
        if k >= 2:
            out_cp[k - 2] = pltpu.async_copy(
                val2d[(k - 2) % 2],
                out_hbm.at[pl.ds((k - 2) * _CR, _CR), pl.ds(c0, _COLS)],
                sem_o[(k - 2) % 2])

    def _make_expand(exp_lin, exp2d):
        def body(q, carry):
            row = q >> 2
            cb = (q & 3) * 128
            ob = q * 128
            for u in range(8):
                c = cb + u * 16
                off = ob + u * 16
                exp2d[row, pl.ds(c, 16)] = exp_lin[pl.ds(off, 16)]
            return carry
        return body

    for t in (_NCHUNK - 2, _NCHUNK - 1):
        g_cp[t].wait()
        out_cp[t - 2].wait()
        lax.fori_loop(0, _CR * _NG // 8,
                      _make_expand(vlin[t % 3], val2d[t % 2]), 0)
        out_cp[t] = pltpu.async_copy(
            val2d[t % 2],
            out_hbm.at[pl.ds(t * _CR, _CR), pl.ds(c0, _COLS)],
            sem_o[t % 2])
    out_cp[_NCHUNK - 2].wait()
    out_cp[_NCHUNK - 1].wait()


def kernel(token_ids, mapping):
    return _lookup(token_ids.T, mapping).T
